# baseline (speedup 1.0000x reference)
import jax
import jax.numpy as jnp
from jax.experimental import pallas as pl
from jax.experimental.pallas import tpu as pltpu

N = 5000
R = 16
K = 20
BI = 200
BC = 1024
NCB = -(-N // BC)
NPAD = NCB * BC
THR = 0.5
NEG = float("-inf")


def _rowmax_kernel(ee_ref, p_ref, pt_ref, pts_ref, out_ref):
    rb = pl.program_id(0)
    cb = pl.program_id(1)
    cols = cb * BC + jax.lax.broadcasted_iota(jnp.int32, (BI, BC), 1)
    s = jnp.where(cols < N, jax.nn.sigmoid(ee_ref[...]), 0.0)
    pure_a = (cb + 1) * BC <= rb * BI
    pure_b = cb * BC >= rb * BI + BI - 1

    @pl.when(cb == 0)
    def _():
        out_ref[...] = jnp.full((BI, R), NEG, jnp.float32)

    @pl.when(pure_a)
    def _():
        for r in range(R):
            m = jnp.max(s * pt_ref[r : r + 1, :], axis=1, keepdims=True)
            out_ref[:, r : r + 1] = jnp.maximum(out_ref[:, r : r + 1], m)

    @pl.when(pure_b)
    def _():
        for r in range(R):
            m = jnp.max(s * pts_ref[r : r + 1, :], axis=1, keepdims=True)
            out_ref[:, r : r + 1] = jnp.maximum(out_ref[:, r : r + 1], m)

    @pl.when(jnp.logical_not(jnp.logical_or(pure_a, pure_b)))
    def _():
        rows = rb * BI + jax.lax.broadcasted_iota(jnp.int32, (BI, BC), 0)
        use_a = cols < rows
        for r in range(R):
            y = jnp.where(use_a, s * pt_ref[r : r + 1, :], s * pts_ref[r : r + 1, :])
            m = jnp.max(y, axis=1, keepdims=True)
            out_ref[:, r : r + 1] = jnp.maximum(out_ref[:, r : r + 1], m)

    @pl.when(cb == NCB - 1)
    def _():
        m = out_ref[...] * p_ref[...]
        out_ref[...] = jnp.where(m > THR, m, NEG)


def _pairsel_kernel(rmt_ref, rows_ref, rels_ref):
    vals = rmt_ref[...]
    lin = (
        jax.lax.broadcasted_iota(jnp.int32, (R, N), 0) * N
        + jax.lax.broadcasted_iota(jnp.int32, (R, N), 1)
    )
    opos = (
        jax.lax.broadcasted_iota(jnp.int32, (8, 128), 0) * 128
        + jax.lax.broadcasted_iota(jnp.int32, (8, 128), 1)
    )
    rows_acc = jnp.zeros((8, 128), jnp.int32)
    rels_acc = jnp.zeros((8, 128), jnp.int32)
    for k in range(K):
        m = jnp.max(vals)
        sel = jnp.min(jnp.where(vals == m, lin, R * N))
        vals = jnp.where(lin == sel, NEG, vals)
        rows_acc = jnp.where(opos == k, sel % N, rows_acc)
        rels_acc = jnp.where(opos == k, sel // N, rels_acc)
    rows_ref[...] = rows_acc
    rels_ref[...] = rels_acc


def _gather_topk_kernel(rows_ref, rels_ref, *refs):
    ee_rows = refs[:K]
    qa_ref, qb_ref, pi_ref, i2_ref = refs[K : K + 4]
    i_ref, r_ref, j_ref, v_ref = refs[K + 4 :]
    cols2 = jax.lax.broadcasted_iota(jnp.int32, (K, N), 1)
    cat = jnp.concatenate([ee_rows[k][0] for k in range(K)], axis=0)
    s = jax.nn.sigmoid(cat)
    sc = s * jnp.where(cols2 < i2_ref[...], qa_ref[...], qb_ref[...])
    sc = sc * pi_ref[...]
    vals = jnp.where(sc > THR, sc, NEG)
    lin = (
        jax.lax.broadcasted_iota(jnp.int32, (K, N), 0) * N
        + jax.lax.broadcasted_iota(jnp.int32, (K, N), 1)
    )
    opos = (
        jax.lax.broadcasted_iota(jnp.int32, (8, 128), 0) * 128
        + jax.lax.broadcasted_iota(jnp.int32, (8, 128), 1)
    )
    i_acc = jnp.zeros((8, 128), jnp.int32)
    r_acc = jnp.zeros((8, 128), jnp.int32)
    j_acc = jnp.zeros((8, 128), jnp.int32)
    v_acc = jnp.full((8, 128), NEG, jnp.float32)
    for t in range(K):
        m = jnp.max(vals)
        sel = jnp.min(jnp.where(vals == m, lin, K * N))
        vals = jnp.where(lin == sel, NEG, vals)
        selk = sel // N
        selc = sel % N
        i_sel = rows_ref[selk]
        r_sel = rels_ref[selk]
        j_sel = selc + jnp.where(selc >= i_sel, 1, 0)
        i_acc = jnp.where(opos == t, i_sel, i_acc)
        r_acc = jnp.where(opos == t, r_sel, r_acc)
        j_acc = jnp.where(opos == t, j_sel, j_acc)
        v_acc = jnp.where(opos == t, m, v_acc)
    i_ref[...] = i_acc
    r_ref[...] = r_acc
    j_ref[...] = j_acc
    v_ref[...] = v_acc


@jax.jit
def kernel(ee_scores, er_scores, entity_labels, relation_types):
    p = jax.nn.sigmoid(er_scores)
    pt = p.T
    pts = jnp.concatenate([pt[:, 1:], jnp.zeros((R, 1), jnp.float32)], axis=1)
    zpad = jnp.zeros((R, NPAD - N), jnp.float32)
    ptp = jnp.concatenate([pt, zpad], axis=1)
    ptsp = jnp.concatenate([pts, zpad], axis=1)

    rowmax = pl.pallas_call(
        _rowmax_kernel,
        grid=(N // BI, NCB),
        in_specs=[
            pl.BlockSpec((BI, BC), lambda b, c: (b, c)),
            pl.BlockSpec((BI, R), lambda b, c: (b, 0)),
            pl.BlockSpec((R, BC), lambda b, c: (0, c)),
            pl.BlockSpec((R, BC), lambda b, c: (0, c)),
        ],
        out_specs=pl.BlockSpec((BI, R), lambda b, c: (b, 0)),
        out_shape=jax.ShapeDtypeStruct((N, R), jnp.float32),
        compiler_params=pltpu.CompilerParams(
            dimension_semantics=("parallel", "arbitrary")
        ),
    )(ee_scores, p, ptp, ptsp)

    rows8, rels8 = pl.pallas_call(
        _pairsel_kernel,
        out_shape=(
            jax.ShapeDtypeStruct((8, 128), jnp.int32),
            jax.ShapeDtypeStruct((8, 128), jnp.int32),
        ),
    )(rowmax.T)
    rows20 = rows8[0, :K]
    rels20 = rels8[0, :K]

    ee3 = ee_scores.reshape(N, 1, N)
    qa20 = jnp.take(pt, rels20, axis=0)
    qb20 = jnp.take(pts, rels20, axis=0)
    pi20 = p[rows20, rels20][:, None]
    i20 = rows20[:, None]
    ee_specs = [
        pl.BlockSpec((1, 1, N), lambda g, rows, rels, k=k: (rows[k], 0, 0))
        for k in range(K)
    ]
    grid_spec = pltpu.PrefetchScalarGridSpec(
        num_scalar_prefetch=2,
        grid=(1,),
        in_specs=ee_specs
        + [
            pl.BlockSpec((K, N), lambda g, rows, rels: (0, 0)),
            pl.BlockSpec((K, N), lambda g, rows, rels: (0, 0)),
            pl.BlockSpec((K, 1), lambda g, rows, rels: (0, 0)),
            pl.BlockSpec((K, 1), lambda g, rows, rels: (0, 0)),
        ],
        out_specs=[
            pl.BlockSpec((8, 128), lambda g, rows, rels: (0, 0)),
            pl.BlockSpec((8, 128), lambda g, rows, rels: (0, 0)),
            pl.BlockSpec((8, 128), lambda g, rows, rels: (0, 0)),
            pl.BlockSpec((8, 128), lambda g, rows, rels: (0, 0)),
        ],
    )
    i8, r8, j8, v8 = pl.pallas_call(
        _gather_topk_kernel,
        grid_spec=grid_spec,
        out_shape=(
            jax.ShapeDtypeStruct((8, 128), jnp.int32),
            jax.ShapeDtypeStruct((8, 128), jnp.int32),
            jax.ShapeDtypeStruct((8, 128), jnp.int32),
            jax.ShapeDtypeStruct((8, 128), jnp.float32),
        ),
    )(rows20, rels20, *([ee3] * K), qa20, qb20, pi20, i20)

    subjects = jnp.take(entity_labels, i8[0, :K])
    relations = jnp.take(relation_types, r8[0, :K])
    objects = jnp.take(entity_labels, j8[0, :K])
    return subjects, relations, objects, v8[0, :K]
